# baseline (speedup 1.0000x reference)
import functools
import numpy as np
import jax
import jax.numpy as jnp
from jax import lax
from jax.experimental import pallas as pl
from jax.experimental.pallas import tpu as pltpu
from jax.experimental.pallas import tpu_sc as plsc

_P_T = 0.1
_P_C = 0.01
_C_SPAN = False
_MASK_T_SPAN = 6
_MASK_C_SPAN = 1
_B, _D, _H, _W = 16, 256, 64, 512
_HW = _H * _W


def _make_span_from_seeds(seeds, span, total):
    inds = []
    for seed in seeds:
        for i in range(int(seed), int(seed) + span):
            if i >= total:
                break
            if i not in inds:
                inds.append(int(i))
    return np.array(inds, dtype=np.int64)


def _make_mask(shape, p, span, rng, use_span):
    mask = np.zeros(shape, dtype=bool)
    for i in range(shape[0]):
        seeds = np.array([], dtype=np.int64)
        while len(seeds) == 0 and p > 0:
            seeds = np.nonzero(rng.random(shape[1]) < p)[0]
        if use_span:
            idx = _make_span_from_seeds(seeds, span, shape[1])
            if idx.size > 0:
                mask[i, idx] = True
        else:
            mask[i, seeds] = True
    return mask


_rng = np.random.default_rng(0)
_MASK_T_NP = _make_mask((_B, _W), _P_T, _MASK_T_SPAN, _rng, True)
_MASK_C_NP = _make_mask((_B, _H), _P_C, _MASK_C_SPAN, _rng, _C_SPAN)

_U = 16


def _build_positions():
    nt_max = 64 * int(_MASK_T_NP.sum(1).max())
    ntp = -(-nt_max // (16 * _U)) * (16 * _U)
    nc_max = int(_MASK_C_NP.sum(1).max())
    ncp = nc_max * _W
    pos_t = np.zeros((_B, ntp), dtype=np.int32)
    pos_c = np.zeros((_B, ncp), dtype=np.int32)
    for b in range(_B):
        ws = np.nonzero(_MASK_T_NP[b])[0].astype(np.int32)
        pt = (np.arange(_H, dtype=np.int32)[:, None] * _W + ws[None, :]).ravel()
        pos_t[b, : pt.size] = pt
        pos_t[b, pt.size:] = pt[0]
        hs = np.nonzero(_MASK_C_NP[b])[0].astype(np.int32)
        pc = (hs[:, None] * _W + np.arange(_W, dtype=np.int32)[None, :]).ravel()
        pos_c[b, : pc.size] = pc
        pos_c[b, pc.size:] = pc[0]
    return pos_t, pos_c


_POS_T_NP, _POS_C_NP = _build_positions()
_NTP = _POS_T_NP.shape[1]
_NCP = _POS_C_NP.shape[1]

_D_PER_W = _D // 2


def _sc_body(x_hbm, pt_hbm, pc_hbm, tr_hbm, cr_hbm, out_hbm,
             buf, pt_v, pc_v, tr_v, cr_v):
    wid = lax.axis_index("s") * 2 + lax.axis_index("c")
    b = wid // 2
    d_base = (wid % 2) * _D_PER_W

    pltpu.sync_copy(pt_hbm.at[b], pt_v)
    pltpu.sync_copy(pc_hbm.at[b], pc_v)
    pltpu.sync_copy(tr_hbm, tr_v)
    pltpu.sync_copy(cr_hbm, cr_v)

    def slice_body(i, carry):
        d = d_base + i
        row = b * _D + d
        pltpu.sync_copy(x_hbm.at[row], buf)
        tv = tr_v[d]
        cv = cr_v[d]

        def t_chunk(j, c2):
            base = j * (16 * _U)
            for u in range(_U):
                idx = pt_v[pl.ds(base + u * 16, 16)]
                plsc.store_scatter(buf, [idx], tv)
            return c2

        lax.fori_loop(0, _NTP // (16 * _U), t_chunk, 0)

        def c_chunk(j, c2):
            base = j * (16 * _U)
            for u in range(_U):
                idx = pc_v[pl.ds(base + u * 16, 16)]
                plsc.store_scatter(buf, [idx], cv)
            return c2

        lax.fori_loop(0, _NCP // (16 * _U), c_chunk, 0)
        pltpu.sync_copy(buf, out_hbm.at[row])
        return carry

    lax.fori_loop(0, _D_PER_W, slice_body, 0)


_sc_call = functools.partial(
    pl.kernel,
    out_type=jax.ShapeDtypeStruct((_B * _D, _HW), jnp.float32),
    mesh=plsc.VectorSubcoreMesh(core_axis_name="c", subcore_axis_name="s"),
    compiler_params=pltpu.CompilerParams(needs_layout_passes=False),
    scratch_types=[
        pltpu.VMEM((_HW,), jnp.float32),
        pltpu.VMEM((_NTP,), jnp.int32),
        pltpu.VMEM((_NCP,), jnp.int32),
        pltpu.VMEM((_D, 16), jnp.float32),
        pltpu.VMEM((_D, 16), jnp.float32),
    ],
)(_sc_body)


def kernel(x, t_mask_replacement, c_mask_replacement):
    mask_t = jnp.asarray(_MASK_T_NP)
    mask_c = jnp.asarray(_MASK_C_NP)
    x2 = x.reshape(_B * _D, _HW)
    tr16 = jnp.tile(t_mask_replacement[:, None], (1, 16))
    cr16 = jnp.tile(c_mask_replacement[:, None], (1, 16))
    out2 = _sc_call(x2, jnp.asarray(_POS_T_NP), jnp.asarray(_POS_C_NP),
                    tr16, cr16)
    return (out2.reshape(_B, _D, _H, _W), x, mask_t, mask_c)


# speedup vs baseline: 1.2175x; 1.2175x over previous
import functools
import numpy as np
import jax
import jax.numpy as jnp
from jax import lax
from jax.experimental import pallas as pl
from jax.experimental.pallas import tpu as pltpu
from jax.experimental.pallas import tpu_sc as plsc

_P_T = 0.1
_P_C = 0.01
_C_SPAN = False
_MASK_T_SPAN = 6
_MASK_C_SPAN = 1
_B, _D, _H, _W = 16, 256, 64, 512


def _make_span_from_seeds(seeds, span, total):
    inds = []
    for seed in seeds:
        for i in range(int(seed), int(seed) + span):
            if i >= total:
                break
            if i not in inds:
                inds.append(int(i))
    return np.array(inds, dtype=np.int64)


def _make_mask(shape, p, span, rng, use_span):
    mask = np.zeros(shape, dtype=bool)
    for i in range(shape[0]):
        seeds = np.array([], dtype=np.int64)
        while len(seeds) == 0 and p > 0:
            seeds = np.nonzero(rng.random(shape[1]) < p)[0]
        if use_span:
            idx = _make_span_from_seeds(seeds, span, shape[1])
            if idx.size > 0:
                mask[i, idx] = True
        else:
            mask[i, seeds] = True
    return mask


_rng = np.random.default_rng(0)
_MASK_T_NP = _make_mask((_B, _W), _P_T, _MASK_T_SPAN, _rng, True)
_MASK_C_NP = _make_mask((_B, _H), _P_C, _MASK_C_SPAN, _rng, _C_SPAN)

_TILED = False


def _off(h, w):
    if _TILED:
        return ((h // 8) * (_W // 128) + w // 128) * 1024 + (h % 8) * 128 + (w % 128)
    return h * _W + w


def _build_lists():
    nmax = int(_MASK_T_NP.sum(1).max())
    nwp = -(-nmax // 16) * 16
    wl = np.zeros((_B, nwp), dtype=np.int32)
    for b in range(_B):
        ws = np.nonzero(_MASK_T_NP[b])[0].astype(np.int32)
        wl[b, : ws.size] = ws
        wl[b, ws.size:] = ws[0]
    ncmax = int(_MASK_C_NP.sum(1).max())
    ncp = ncmax * _W
    ihc = np.zeros((_B, ncp), dtype=np.int32)
    iwc = np.zeros((_B, ncp), dtype=np.int32)
    for b in range(_B):
        hs = np.nonzero(_MASK_C_NP[b])[0].astype(np.int32)
        ih = np.repeat(hs, _W)
        iw = np.tile(np.arange(_W, dtype=np.int32), hs.size)
        ihc[b, : ih.size] = ih
        iwc[b, : iw.size] = iw
        ihc[b, ih.size:] = ih[0]
        iwc[b, iw.size:] = iw[0]
    return wl, ihc, iwc


_WL_NP, _IHC_NP, _IWC_NP = _build_lists()
_NWP = _WL_NP.shape[1]
_NCP = _IHC_NP.shape[1]

_D_PER_W = _D // 2
_NSL = _D_PER_W + 1
_NGRP = _NSL // 3


def _scatter_slice(buf, wl_v, ihc_v, iwc_v, tv, cv):
    def h_body(h, c):
        ih = jnp.full((16,), h, jnp.int32)
        for u in range(_NWP // 16):
            iw = wl_v[pl.ds(u * 16, 16)]
            plsc.store_scatter(buf, [ih, iw], tv)
        return c

    lax.fori_loop(0, _H, h_body, 0)
    for u in range(_NCP // 16):
        ih = ihc_v[pl.ds(u * 16, 16)]
        iw = iwc_v[pl.ds(u * 16, 16)]
        plsc.store_scatter(buf, [ih, iw], cv)


def _sc_body(x_hbm, wl_hbm, ihc_hbm, iwc_hbm, tr_hbm, cr_hbm, out_hbm,
             b0, b1, b2, wl_v, ihc_v, iwc_v, tr_v, cr_v,
             si0, si1, si2, so0, so1, so2):
    bufs = (b0, b1, b2)
    sin = (si0, si1, si2)
    sout = (so0, so1, so2)
    wid = lax.axis_index("s") * 2 + lax.axis_index("c")
    b = wid // 2
    d_base = (wid % 2) * _D_PER_W
    row0 = b * _D + d_base

    pltpu.sync_copy(wl_hbm.at[b], wl_v)
    pltpu.sync_copy(ihc_hbm.at[b], ihc_v)
    pltpu.sync_copy(iwc_hbm.at[b], iwc_v)
    pltpu.sync_copy(tr_hbm.at[pl.ds(d_base * 16, _D_PER_W * 16)], tr_v)
    pltpu.sync_copy(cr_hbm.at[pl.ds(d_base * 16, _D_PER_W * 16)], cr_v)

    def srow(s):
        return row0 + jnp.minimum(s, _D_PER_W - 1)

    pltpu.async_copy(x_hbm.at[srow(0)], bufs[0], sin[0])
    pltpu.async_copy(x_hbm.at[srow(1)], bufs[1], sin[1])

    def group(g, carry):
        for k in range(3):
            s = 3 * g + k
            kin = (k + 2) % 3
            def prefetch():
                pltpu.make_async_copy(x_hbm.at[row0], bufs[kin], sout[kin]).wait()
                pltpu.async_copy(x_hbm.at[srow(s + 2)], bufs[kin], sin[kin])

            if k == 0:
                @pl.when(g > 0)
                def _():
                    prefetch()

                @pl.when(g == 0)
                def _():
                    pltpu.async_copy(x_hbm.at[srow(2)], bufs[2], sin[2])
            elif k == 1:
                @pl.when(g < _NGRP - 1)
                def _():
                    prefetch()
            else:
                @pl.when(g < _NGRP - 1)
                def _():
                    prefetch()

            pltpu.make_async_copy(x_hbm.at[row0], bufs[k], sin[k]).wait()
            d = jnp.minimum(s, _D_PER_W - 1)
            _scatter_slice(bufs[k], wl_v, ihc_v, iwc_v,
                           tr_v[pl.ds(d * 16, 16)], cr_v[pl.ds(d * 16, 16)])
            pltpu.async_copy(bufs[k], out_hbm.at[srow(s)], sout[k])
        return carry

    lax.fori_loop(0, _NGRP, group, 0)
    for k in range(3):
        pltpu.make_async_copy(x_hbm.at[row0], bufs[k], sout[k]).wait()


_sc_call = functools.partial(
    pl.kernel,
    out_type=jax.ShapeDtypeStruct((_B * _D, _H, _W), jnp.float32),
    mesh=plsc.VectorSubcoreMesh(core_axis_name="c", subcore_axis_name="s"),
    compiler_params=pltpu.CompilerParams(needs_layout_passes=False),
    scratch_types=[
        pltpu.VMEM((_H, _W), jnp.float32),
        pltpu.VMEM((_H, _W), jnp.float32),
        pltpu.VMEM((_H, _W), jnp.float32),
        pltpu.VMEM((_NWP,), jnp.int32),
        pltpu.VMEM((_NCP,), jnp.int32),
        pltpu.VMEM((_NCP,), jnp.int32),
        pltpu.VMEM((_D_PER_W * 16,), jnp.float32),
        pltpu.VMEM((_D_PER_W * 16,), jnp.float32),
        pltpu.SemaphoreType.DMA,
        pltpu.SemaphoreType.DMA,
        pltpu.SemaphoreType.DMA,
        pltpu.SemaphoreType.DMA,
        pltpu.SemaphoreType.DMA,
        pltpu.SemaphoreType.DMA,
    ],
)(_sc_body)


def kernel(x, t_mask_replacement, c_mask_replacement):
    mask_t = jnp.asarray(_MASK_T_NP)
    mask_c = jnp.asarray(_MASK_C_NP)
    x3 = x.reshape(_B * _D, _H, _W)
    tr16 = jnp.tile(t_mask_replacement[:, None], (1, 16)).reshape(-1)
    cr16 = jnp.tile(c_mask_replacement[:, None], (1, 16)).reshape(-1)
    out3 = _sc_call(x3, jnp.asarray(_WL_NP), jnp.asarray(_IHC_NP),
                    jnp.asarray(_IWC_NP), tr16, cr16)
    return (out3.reshape(_B, _D, _H, _W), x, mask_t, mask_c)


# speedup vs baseline: 1.9290x; 1.5844x over previous
import functools
import numpy as np
import jax
import jax.numpy as jnp
from jax import lax
from jax.experimental import pallas as pl
from jax.experimental.pallas import tpu as pltpu
from jax.experimental.pallas import tpu_sc as plsc

_P_T = 0.1
_P_C = 0.01
_C_SPAN = False
_MASK_T_SPAN = 6
_MASK_C_SPAN = 1
_B, _D, _H, _W = 16, 256, 64, 512


def _make_span_from_seeds(seeds, span, total):
    inds = []
    for seed in seeds:
        for i in range(int(seed), int(seed) + span):
            if i >= total:
                break
            if i not in inds:
                inds.append(int(i))
    return np.array(inds, dtype=np.int64)


def _make_mask(shape, p, span, rng, use_span):
    mask = np.zeros(shape, dtype=bool)
    for i in range(shape[0]):
        seeds = np.array([], dtype=np.int64)
        while len(seeds) == 0 and p > 0:
            seeds = np.nonzero(rng.random(shape[1]) < p)[0]
        if use_span:
            idx = _make_span_from_seeds(seeds, span, shape[1])
            if idx.size > 0:
                mask[i, idx] = True
        else:
            mask[i, seeds] = True
    return mask


_rng = np.random.default_rng(0)
_MASK_T_NP = _make_mask((_B, _W), _P_T, _MASK_T_SPAN, _rng, True)
_MASK_C_NP = _make_mask((_B, _H), _P_C, _MASK_C_SPAN, _rng, _C_SPAN)

_TILED = False


def _off(h, w):
    if _TILED:
        return ((h // 8) * (_W // 128) + w // 128) * 1024 + (h % 8) * 128 + (w % 128)
    return h * _W + w


def _build_lists():
    wl = _MASK_T_NP.astype(np.int32)
    ncmax = int(_MASK_C_NP.sum(1).max())
    ncp = ncmax * _W
    ihc = np.zeros((_B, ncp), dtype=np.int32)
    iwc = np.zeros((_B, ncp), dtype=np.int32)
    for b in range(_B):
        hs = np.nonzero(_MASK_C_NP[b])[0].astype(np.int32)
        ih = np.repeat(hs, _W)
        iw = np.tile(np.arange(_W, dtype=np.int32), hs.size)
        ihc[b, : ih.size] = ih
        iwc[b, : iw.size] = iw
        ihc[b, ih.size:] = ih[0]
        iwc[b, iw.size:] = iw[0]
    return wl, ihc, iwc


_WL_NP, _IHC_NP, _IWC_NP = _build_lists()
_NCP = _IHC_NP.shape[1]

_D_PER_W = _D // 2
_NSL = _D_PER_W + 1
_NGRP = _NSL // 3


def _apply_slice(buf, masks, ihc_v, iwc_v, tv, cv):
    def h_body(h, c):
        for u in range(_W // 16):
            xv = buf[h, pl.ds(u * 16, 16)]
            buf[h, pl.ds(u * 16, 16)] = jnp.where(masks[u], tv, xv)
        return c

    lax.fori_loop(0, _H, h_body, 0)
    for u in range(_NCP // 16):
        ih = ihc_v[pl.ds(u * 16, 16)]
        iw = iwc_v[pl.ds(u * 16, 16)]
        plsc.store_scatter(buf, [ih, iw], cv)


def _sc_body(x_hbm, wl_hbm, ihc_hbm, iwc_hbm, tr_hbm, cr_hbm, out_hbm,
             b0, b1, b2, wl_v, ihc_v, iwc_v, tr_v, cr_v,
             si0, si1, si2, so0, so1, so2):
    bufs = (b0, b1, b2)
    sin = (si0, si1, si2)
    sout = (so0, so1, so2)
    wid = lax.axis_index("s") * 2 + lax.axis_index("c")
    b = wid // 2
    d_base = (wid % 2) * _D_PER_W
    row0 = b * _D + d_base

    pltpu.sync_copy(wl_hbm.at[b], wl_v)
    masks = [wl_v[pl.ds(u * 16, 16)] != 0 for u in range(_W // 16)]
    pltpu.sync_copy(ihc_hbm.at[b], ihc_v)
    pltpu.sync_copy(iwc_hbm.at[b], iwc_v)
    pltpu.sync_copy(tr_hbm.at[pl.ds(d_base * 16, _D_PER_W * 16)], tr_v)
    pltpu.sync_copy(cr_hbm.at[pl.ds(d_base * 16, _D_PER_W * 16)], cr_v)

    def srow(s):
        return row0 + jnp.minimum(s, _D_PER_W - 1)

    pltpu.async_copy(x_hbm.at[srow(0)], bufs[0], sin[0])
    pltpu.async_copy(x_hbm.at[srow(1)], bufs[1], sin[1])

    def group(g, carry):
        for k in range(3):
            s = 3 * g + k
            kin = (k + 2) % 3
            def prefetch():
                pltpu.make_async_copy(x_hbm.at[row0], bufs[kin], sout[kin]).wait()
                pltpu.async_copy(x_hbm.at[srow(s + 2)], bufs[kin], sin[kin])

            if k == 0:
                @pl.when(g > 0)
                def _():
                    prefetch()

                @pl.when(g == 0)
                def _():
                    pltpu.async_copy(x_hbm.at[srow(2)], bufs[2], sin[2])
            elif k == 1:
                @pl.when(g < _NGRP - 1)
                def _():
                    prefetch()
            else:
                @pl.when(g < _NGRP - 1)
                def _():
                    prefetch()

            pltpu.make_async_copy(x_hbm.at[row0], bufs[k], sin[k]).wait()
            d = jnp.minimum(s, _D_PER_W - 1)
            _apply_slice(bufs[k], masks, ihc_v, iwc_v,
                         tr_v[pl.ds(d * 16, 16)], cr_v[pl.ds(d * 16, 16)])
            pltpu.async_copy(bufs[k], out_hbm.at[srow(s)], sout[k])
        return carry

    lax.fori_loop(0, _NGRP, group, 0)
    for k in range(3):
        pltpu.make_async_copy(x_hbm.at[row0], bufs[k], sout[k]).wait()


_sc_call = functools.partial(
    pl.kernel,
    out_type=jax.ShapeDtypeStruct((_B * _D, _H, _W), jnp.float32),
    mesh=plsc.VectorSubcoreMesh(core_axis_name="c", subcore_axis_name="s"),
    compiler_params=pltpu.CompilerParams(needs_layout_passes=False),
    scratch_types=[
        pltpu.VMEM((_H, _W), jnp.float32),
        pltpu.VMEM((_H, _W), jnp.float32),
        pltpu.VMEM((_H, _W), jnp.float32),
        pltpu.VMEM((_W,), jnp.int32),
        pltpu.VMEM((_NCP,), jnp.int32),
        pltpu.VMEM((_NCP,), jnp.int32),
        pltpu.VMEM((_D_PER_W * 16,), jnp.float32),
        pltpu.VMEM((_D_PER_W * 16,), jnp.float32),
        pltpu.SemaphoreType.DMA,
        pltpu.SemaphoreType.DMA,
        pltpu.SemaphoreType.DMA,
        pltpu.SemaphoreType.DMA,
        pltpu.SemaphoreType.DMA,
        pltpu.SemaphoreType.DMA,
    ],
)(_sc_body)


def kernel(x, t_mask_replacement, c_mask_replacement):
    mask_t = jnp.asarray(_MASK_T_NP)
    mask_c = jnp.asarray(_MASK_C_NP)
    x3 = x.reshape(_B * _D, _H, _W)
    tr16 = jnp.tile(t_mask_replacement[:, None], (1, 16)).reshape(-1)
    cr16 = jnp.tile(c_mask_replacement[:, None], (1, 16)).reshape(-1)
    out3 = _sc_call(x3, jnp.asarray(_WL_NP), jnp.asarray(_IHC_NP),
                    jnp.asarray(_IWC_NP), tr16, cr16)
    return (out3.reshape(_B, _D, _H, _W), x, mask_t, mask_c)


# speedup vs baseline: 2.4848x; 1.2882x over previous
import functools
import numpy as np
import jax
import jax.numpy as jnp
from jax import lax
from jax.experimental import pallas as pl
from jax.experimental.pallas import tpu as pltpu

_P_T = 0.1
_P_C = 0.01
_C_SPAN = False
_MASK_T_SPAN = 6
_MASK_C_SPAN = 1
_B, _D, _H, _W = 16, 256, 64, 512


def _make_span_from_seeds(seeds, span, total):
    inds = []
    for seed in seeds:
        for i in range(int(seed), int(seed) + span):
            if i >= total:
                break
            if i not in inds:
                inds.append(int(i))
    return np.array(inds, dtype=np.int64)


def _make_mask(shape, p, span, rng, use_span):
    mask = np.zeros(shape, dtype=bool)
    for i in range(shape[0]):
        seeds = np.array([], dtype=np.int64)
        while len(seeds) == 0 and p > 0:
            seeds = np.nonzero(rng.random(shape[1]) < p)[0]
        if use_span:
            idx = _make_span_from_seeds(seeds, span, shape[1])
            if idx.size > 0:
                mask[i, idx] = True
        else:
            mask[i, seeds] = True
    return mask


_rng = np.random.default_rng(0)
_MASK_T_NP = _make_mask((_B, _W), _P_T, _MASK_T_SPAN, _rng, True)
_MASK_C_NP = _make_mask((_B, _H), _P_C, _MASK_C_SPAN, _rng, _C_SPAN)

_K = 8
_RPC = 8
_NR = _B * _D
_NCH = _NR // _RPC
_SPB = _D // _RPC


def _ring_kernel(x_ref, mt_ref, mc_ref, tr_ref, cr_ref, o_ref, *rest):
    bufs = rest[:_K]
    sin = rest[_K:2 * _K]
    sout = rest[2 * _K:3 * _K]

    def start_in(j, s):
        pltpu.make_async_copy(x_ref.at[pl.ds(s * _RPC, _RPC)], bufs[j], sin[j]).start()

    def wait_in(j):
        pltpu.make_async_copy(x_ref.at[pl.ds(0, _RPC)], bufs[j], sin[j]).wait()

    def start_out(j, s):
        pltpu.make_async_copy(bufs[j], o_ref.at[pl.ds(s * _RPC, _RPC)], sout[j]).start()

    def wait_out(j):
        pltpu.make_async_copy(bufs[j], o_ref.at[pl.ds(0, _RPC)], sout[j]).wait()

    for j in range(_K - 1):
        start_in(j, j)

    def group(g, carry):
        for k in range(_K):
            s = g * _K + k
            j = (k + _K - 1) % _K
            if k == 0:
                @pl.when(g == 0)
                def _():
                    start_in(j, _K - 1)

                @pl.when(g > 0)
                def _():
                    wait_out(j)
                    start_in(j, s + _K - 1)
            else:
                @pl.when(s + _K - 1 < _NCH)
                def _():
                    wait_out(j)
                    start_in(j, s + _K - 1)

            wait_in(k)
            b = s // _SPB
            dmod = (s % _SPB) * _RPC
            mtb = mt_ref[pl.ds(b, 1), :][:, None, :]
            mcb = mc_ref[pl.ds(b, 1), :][:, :, None]
            tr8 = tr_ref[pl.ds(dmod, _RPC), :][:, :, None]
            cr8 = cr_ref[pl.ds(dmod, _RPC), :][:, :, None]
            xv = bufs[k][...]
            bufs[k][...] = jnp.where(mcb != 0, cr8,
                                     jnp.where(mtb != 0, tr8, xv))
            start_out(k, s)
        return carry

    lax.fori_loop(0, _NCH // _K, group, 0)
    for k in range(_K):
        wait_out(k)


_tc_call = pl.pallas_call(
    _ring_kernel,
    in_specs=[
        pl.BlockSpec(memory_space=pltpu.MemorySpace.HBM),
        pl.BlockSpec(memory_space=pltpu.VMEM),
        pl.BlockSpec(memory_space=pltpu.VMEM),
        pl.BlockSpec(memory_space=pltpu.VMEM),
        pl.BlockSpec(memory_space=pltpu.VMEM),
    ],
    out_specs=pl.BlockSpec(memory_space=pltpu.MemorySpace.HBM),
    out_shape=jax.ShapeDtypeStruct((_NR, _H, _W), jnp.float32),
    scratch_shapes=(
        [pltpu.VMEM((_RPC, _H, _W), jnp.float32) for _ in range(_K)]
        + [pltpu.SemaphoreType.DMA] * (2 * _K)
    ),
)


def kernel(x, t_mask_replacement, c_mask_replacement):
    mask_t = jnp.asarray(_MASK_T_NP)
    mask_c = jnp.asarray(_MASK_C_NP)
    x3 = x.reshape(_NR, _H, _W)
    mt_f = mask_t.astype(jnp.float32)
    mc_f = mask_c.astype(jnp.float32)
    tr = t_mask_replacement.reshape(_D, 1)
    cr = c_mask_replacement.reshape(_D, 1)
    out3 = _tc_call(x3, mt_f, mc_f, tr, cr)
    return (out3.reshape(_B, _D, _H, _W), x, mask_t, mask_c)


# speedup vs baseline: 3.5482x; 1.4279x over previous
import numpy as np
import jax
import jax.numpy as jnp
from jax.experimental import pallas as pl

_P_T = 0.1
_P_C = 0.01
_C_SPAN = False
_MASK_T_SPAN = 6
_MASK_C_SPAN = 1
_B, _D, _H, _W = 16, 256, 64, 512


def _make_span_from_seeds(seeds, span, total):
    inds = []
    for seed in seeds:
        for i in range(int(seed), int(seed) + span):
            if i >= total:
                break
            if i not in inds:
                inds.append(int(i))
    return np.array(inds, dtype=np.int64)


def _make_mask(shape, p, span, rng, use_span):
    mask = np.zeros(shape, dtype=bool)
    for i in range(shape[0]):
        seeds = np.array([], dtype=np.int64)
        while len(seeds) == 0 and p > 0:
            seeds = np.nonzero(rng.random(shape[1]) < p)[0]
        if use_span:
            idx = _make_span_from_seeds(seeds, span, shape[1])
            if idx.size > 0:
                mask[i, idx] = True
        else:
            mask[i, seeds] = True
    return mask


_rng = np.random.default_rng(0)
_MASK_T_NP = _make_mask((_B, _W), _P_T, _MASK_T_SPAN, _rng, True)
_MASK_C_NP = _make_mask((_B, _H), _P_C, _MASK_C_SPAN, _rng, _C_SPAN)

_DBLK = 32


def _select_kernel(x_ref, mt_ref, mc_ref, tr_ref, cr_ref, o_ref):
    x = x_ref[...]
    mt = mt_ref[...][:, :, None, :]
    mc = mc_ref[...][:, :, :, None]
    tr = tr_ref[...][None, :, :, None]
    cr = cr_ref[...][None, :, :, None]
    o_ref[...] = jnp.where(mc != 0, cr, jnp.where(mt != 0, tr, x))


def kernel(x, t_mask_replacement, c_mask_replacement):
    mask_t = jnp.asarray(_MASK_T_NP)
    mask_c = jnp.asarray(_MASK_C_NP)
    mt_f = mask_t.astype(jnp.float32).reshape(_B, 1, _W)
    mc_f = mask_c.astype(jnp.float32).reshape(_B, 1, _H)
    tr = t_mask_replacement.reshape(_D, 1)
    cr = c_mask_replacement.reshape(_D, 1)

    grid = (_B, _D // _DBLK)
    out = pl.pallas_call(
        _select_kernel,
        grid=grid,
        in_specs=[
            pl.BlockSpec((1, _DBLK, _H, _W), lambda b, d: (b, d, 0, 0)),
            pl.BlockSpec((1, 1, _W), lambda b, d: (b, 0, 0)),
            pl.BlockSpec((1, 1, _H), lambda b, d: (b, 0, 0)),
            pl.BlockSpec((_DBLK, 1), lambda b, d: (d, 0)),
            pl.BlockSpec((_DBLK, 1), lambda b, d: (d, 0)),
        ],
        out_specs=pl.BlockSpec((1, _DBLK, _H, _W), lambda b, d: (b, d, 0, 0)),
        out_shape=jax.ShapeDtypeStruct((_B, _D, _H, _W), jnp.float32),
    )(x, mt_f, mc_f, tr, cr)

    return (out, x, mask_t, mask_c)


# speedup vs baseline: 3.5637x; 1.0044x over previous
import numpy as np
import jax
import jax.numpy as jnp
from jax.experimental import pallas as pl

_P_T = 0.1
_P_C = 0.01
_C_SPAN = False
_MASK_T_SPAN = 6
_MASK_C_SPAN = 1
_B, _D, _H, _W = 16, 256, 64, 512


def _make_span_from_seeds(seeds, span, total):
    inds = []
    for seed in seeds:
        for i in range(int(seed), int(seed) + span):
            if i >= total:
                break
            if i not in inds:
                inds.append(int(i))
    return np.array(inds, dtype=np.int64)


def _make_mask(shape, p, span, rng, use_span):
    mask = np.zeros(shape, dtype=bool)
    for i in range(shape[0]):
        seeds = np.array([], dtype=np.int64)
        while len(seeds) == 0 and p > 0:
            seeds = np.nonzero(rng.random(shape[1]) < p)[0]
        if use_span:
            idx = _make_span_from_seeds(seeds, span, shape[1])
            if idx.size > 0:
                mask[i, idx] = True
        else:
            mask[i, seeds] = True
    return mask


_rng = np.random.default_rng(0)
_MASK_T_NP = _make_mask((_B, _W), _P_T, _MASK_T_SPAN, _rng, True)
_MASK_C_NP = _make_mask((_B, _H), _P_C, _MASK_C_SPAN, _rng, _C_SPAN)

_DBLK = 64


def _select_kernel(x_ref, mt_ref, mc_ref, tr_ref, cr_ref, o_ref):
    x = x_ref[...]
    mt = mt_ref[...][:, :, None, :]
    mc = mc_ref[...][:, :, :, None]
    tr = tr_ref[...][None, :, :, None]
    cr = cr_ref[...][None, :, :, None]
    o_ref[...] = jnp.where(mc != 0, cr, jnp.where(mt != 0, tr, x))


def kernel(x, t_mask_replacement, c_mask_replacement):
    mask_t = jnp.asarray(_MASK_T_NP)
    mask_c = jnp.asarray(_MASK_C_NP)
    mt_f = mask_t.astype(jnp.float32).reshape(_B, 1, _W)
    mc_f = mask_c.astype(jnp.float32).reshape(_B, 1, _H)
    tr = t_mask_replacement.reshape(_D, 1)
    cr = c_mask_replacement.reshape(_D, 1)

    grid = (_B, _D // _DBLK)
    out = pl.pallas_call(
        _select_kernel,
        grid=grid,
        in_specs=[
            pl.BlockSpec((1, _DBLK, _H, _W), lambda b, d: (b, d, 0, 0)),
            pl.BlockSpec((1, 1, _W), lambda b, d: (b, 0, 0)),
            pl.BlockSpec((1, 1, _H), lambda b, d: (b, 0, 0)),
            pl.BlockSpec((_DBLK, 1), lambda b, d: (d, 0)),
            pl.BlockSpec((_DBLK, 1), lambda b, d: (d, 0)),
        ],
        out_specs=pl.BlockSpec((1, _DBLK, _H, _W), lambda b, d: (b, d, 0, 0)),
        out_shape=jax.ShapeDtypeStruct((_B, _D, _H, _W), jnp.float32),
    )(x, mt_f, mc_f, tr, cr)

    return (out, x, mask_t, mask_c)
